# Initial kernel scaffold; baseline (speedup 1.0000x reference)
import functools

import jax
import jax.numpy as jnp
from jax import lax
from jax.experimental import pallas as pl
from jax.experimental.pallas import tpu as pltpu
from jax.experimental.pallas import tpu_sc as plsc

N = 10000
E = 320000
D = 128

HALF = N // 2
TRASH = 5120
ACC_ROWS = 5128
BLK = 128
N_SUB = 16
EPT = 20096
NBLK = EPT // BLK
E_PAD = N_SUB * EPT


def _spmm_body(dst_hbm, col_hbm, w_hbm, feat_hbm, out_hbm,
               dst_v, col_v, w_v, rows_v, acc, sem):
    c = lax.axis_index("c")
    s = lax.axis_index("s")
    lo = c * HALF

    def _zero_row(r, _):
        for j in range(D // 16):
            rows_v[r, pl.ds(j * 16, 16)] = jnp.zeros((16,), jnp.float32)
        return 0
    lax.fori_loop(0, BLK, _zero_row, 0)

    zbase = s * 320
    pltpu.sync_copy(rows_v, acc.at[pl.ds(zbase, 128)])
    pltpu.sync_copy(rows_v, acc.at[pl.ds(zbase + 128, 128)])
    pltpu.sync_copy(rows_v.at[pl.ds(0, 64)], acc.at[pl.ds(zbase + 256, 64)])

    @pl.when(s == 0)
    def _():
        pltpu.sync_copy(rows_v.at[pl.ds(0, 8)], acc.at[pl.ds(5120, 8)])

    plsc.subcore_barrier()

    tile_base = s * EPT

    def _block(b, _):
        ebase = tile_base + b * BLK
        pltpu.sync_copy(dst_hbm.at[pl.ds(ebase, BLK)], dst_v)
        pltpu.sync_copy(col_hbm.at[pl.ds(ebase, BLK)], col_v)
        pltpu.sync_copy(w_hbm.at[pl.ds(ebase, BLK)], w_v)
        pltpu.async_copy(feat_hbm.at[col_v], rows_v, sem).wait()

        for k in range(BLK // 16):
            dv = dst_v[pl.ds(k * 16, 16)]
            inr = (dv >= lo) & (dv < lo + HALF)
            dst_v[pl.ds(k * 16, 16)] = jnp.where(inr, dv - lo, TRASH)

        def _scale(i, _):
            w = w_v[i]
            for j in range(D // 16):
                rows_v[i, pl.ds(j * 16, 16)] = (
                    rows_v[i, pl.ds(j * 16, 16)] * w)
            return 0
        lax.fori_loop(0, BLK, _scale, 0)

        pltpu.sync_copy(rows_v, acc.at[dst_v], add=True)
        return 0

    lax.fori_loop(0, NBLK, _block, 0)

    plsc.subcore_barrier()

    @pl.when(s < 15)
    def _():
        base = s * 320
        pltpu.sync_copy(acc.at[pl.ds(base, 128)], rows_v)
        pltpu.sync_copy(rows_v, out_hbm.at[pl.ds(lo + base, 128)])
        pltpu.sync_copy(acc.at[pl.ds(base + 128, 128)], rows_v)
        pltpu.sync_copy(rows_v, out_hbm.at[pl.ds(lo + base + 128, 128)])
        pltpu.sync_copy(acc.at[pl.ds(base + 256, 64)], rows_v.at[pl.ds(0, 64)])
        pltpu.sync_copy(rows_v.at[pl.ds(0, 64)],
                        out_hbm.at[pl.ds(lo + base + 256, 64)])

    @pl.when(s == 15)
    def _():
        pltpu.sync_copy(acc.at[pl.ds(4800, 128)], rows_v)
        pltpu.sync_copy(rows_v, out_hbm.at[pl.ds(lo + 4800, 128)])
        pltpu.sync_copy(acc.at[pl.ds(4928, 72)], rows_v.at[pl.ds(0, 72)])
        pltpu.sync_copy(rows_v.at[pl.ds(0, 72)],
                        out_hbm.at[pl.ds(lo + 4928, 72)])


@jax.jit
def _spmm(dst, col, w, feat):
    mesh = plsc.VectorSubcoreMesh(core_axis_name="c", subcore_axis_name="s")
    run = functools.partial(
        pl.kernel,
        mesh=mesh,
        out_type=jax.ShapeDtypeStruct((N, D), jnp.float32),
        scratch_types=[
            pltpu.VMEM((BLK,), jnp.int32),
            pltpu.VMEM((BLK,), jnp.int32),
            pltpu.VMEM((BLK,), jnp.float32),
            pltpu.VMEM((BLK, D), jnp.float32),
            pltpu.VMEM_SHARED((ACC_ROWS, D), jnp.float32),
            pltpu.SemaphoreType.DMA,
        ],
    )(_spmm_body)
    return run(dst, col, w, feat)


def kernel(edge_index, adj_values, feat):
    dst = edge_index[0].astype(jnp.int32)
    col = edge_index[1].astype(jnp.int32)
    pad = E_PAD - E
    dst = jnp.pad(dst, (0, pad))
    col = jnp.pad(col, (0, pad))
    w = jnp.pad(adj_values, (0, pad))
    return _spmm(dst, col, w, feat)


# speedup vs baseline: 2.8014x; 2.8014x over previous
import functools

import jax
import jax.numpy as jnp
from jax import lax
from jax.experimental import pallas as pl
from jax.experimental.pallas import tpu as pltpu
from jax.experimental.pallas import tpu_sc as plsc

N = 10000
E = 320000
D = 128

HALF = N // 2
TRASH = 5120
ACC_ROWS = 5128
BLK = 128
N_SUB = 16
EPT = 20096
NBLK = EPT // BLK
E_PAD = N_SUB * EPT


def _spmm_body(dst_hbm, col_hbm, w_hbm, feat_hbm, out_hbm,
               dst_v, col_v, w_v, rows_v, acc, sem):
    c = lax.axis_index("c")
    s = lax.axis_index("s")
    lo = c * HALF

    def _zero_row(r, _):
        for j in range(D // 16):
            rows_v[r, pl.ds(j * 16, 16)] = jnp.zeros((16,), jnp.float32)
        return 0
    lax.fori_loop(0, BLK, _zero_row, 0)

    zbase = s * 320
    pltpu.sync_copy(rows_v, acc.at[pl.ds(zbase, 128)])
    pltpu.sync_copy(rows_v, acc.at[pl.ds(zbase + 128, 128)])
    pltpu.sync_copy(rows_v.at[pl.ds(0, 64)], acc.at[pl.ds(zbase + 256, 64)])

    @pl.when(s == 0)
    def _():
        pltpu.sync_copy(rows_v.at[pl.ds(0, 8)], acc.at[pl.ds(5120, 8)])

    plsc.subcore_barrier()

    tile_base = s * EPT

    def _block(b, _):
        ebase = tile_base + b * BLK
        pltpu.sync_copy(dst_hbm.at[pl.ds(ebase, BLK)], dst_v)
        pltpu.sync_copy(col_hbm.at[pl.ds(ebase, BLK)], col_v)
        pltpu.sync_copy(w_hbm.at[pl.ds(ebase, BLK)], w_v)
        pltpu.async_copy(feat_hbm.at[col_v], rows_v, sem).wait()

        for k in range(BLK // 16):
            dv = dst_v[pl.ds(k * 16, 16)]
            inr = (dv >= lo) & (dv < lo + HALF)
            dst_v[pl.ds(k * 16, 16)] = jnp.where(inr, dv - lo, TRASH)

        def _scale(g, _):
            w16 = w_v[pl.ds(g * 16, 16)]
            for i in range(16):
                e = g * 16 + i
                w = jnp.broadcast_to(w16[i], (16,))
                for j in range(D // 16):
                    rows_v[e, pl.ds(j * 16, 16)] = (
                        rows_v[e, pl.ds(j * 16, 16)] * w)
            return 0
        lax.fori_loop(0, BLK // 16, _scale, 0)

        pltpu.sync_copy(rows_v, acc.at[dst_v], add=True)
        return 0

    lax.fori_loop(0, NBLK, _block, 0)

    plsc.subcore_barrier()

    @pl.when(s < 15)
    def _():
        base = s * 320
        pltpu.sync_copy(acc.at[pl.ds(base, 128)], rows_v)
        pltpu.sync_copy(rows_v, out_hbm.at[pl.ds(lo + base, 128)])
        pltpu.sync_copy(acc.at[pl.ds(base + 128, 128)], rows_v)
        pltpu.sync_copy(rows_v, out_hbm.at[pl.ds(lo + base + 128, 128)])
        pltpu.sync_copy(acc.at[pl.ds(base + 256, 64)], rows_v.at[pl.ds(0, 64)])
        pltpu.sync_copy(rows_v.at[pl.ds(0, 64)],
                        out_hbm.at[pl.ds(lo + base + 256, 64)])

    @pl.when(s == 15)
    def _():
        pltpu.sync_copy(acc.at[pl.ds(4800, 128)], rows_v)
        pltpu.sync_copy(rows_v, out_hbm.at[pl.ds(lo + 4800, 128)])
        pltpu.sync_copy(acc.at[pl.ds(4928, 72)], rows_v.at[pl.ds(0, 72)])
        pltpu.sync_copy(rows_v.at[pl.ds(0, 72)],
                        out_hbm.at[pl.ds(lo + 4928, 72)])


@jax.jit
def _spmm(dst, col, w, feat):
    mesh = plsc.VectorSubcoreMesh(core_axis_name="c", subcore_axis_name="s")
    run = functools.partial(
        pl.kernel,
        mesh=mesh,
        out_type=jax.ShapeDtypeStruct((N, D), jnp.float32),
        scratch_types=[
            pltpu.VMEM((BLK,), jnp.int32),
            pltpu.VMEM((BLK,), jnp.int32),
            pltpu.VMEM((BLK,), jnp.float32),
            pltpu.VMEM((BLK, D), jnp.float32),
            pltpu.VMEM_SHARED((ACC_ROWS, D), jnp.float32),
            pltpu.SemaphoreType.DMA,
        ],
    )(_spmm_body)
    return run(dst, col, w, feat)


def kernel(edge_index, adj_values, feat):
    dst = edge_index[0].astype(jnp.int32)
    col = edge_index[1].astype(jnp.int32)
    pad = E_PAD - E
    dst = jnp.pad(dst, (0, pad))
    col = jnp.pad(col, (0, pad))
    w = jnp.pad(adj_values, (0, pad))
    return _spmm(dst, col, w, feat)
